# Initial kernel scaffold; baseline (speedup 1.0000x reference)
import jax
import jax.numpy as jnp
from jax.experimental import pallas as pl


def kernel(idx, tables, W1, b1, W2, b2):
    raise NotImplementedError("write your pallas kernel here")


# speedup vs baseline: 14.5285x; 14.5285x over previous
import functools

import jax
import jax.numpy as jnp
from jax.experimental import pallas as pl
from jax.experimental.pallas import tpu as pltpu
from jax.experimental.pallas import tpu_sc as plsc

B = 16384
F = 26
V = 1000
D = 64
H = 128
DP = 128

GW = 128
BLK = 256


def _gather_rows(table, flat_idx):
    n = flat_idx.shape[1]
    w = table.shape[1]
    mesh = plsc.VectorSubcoreMesh(core_axis_name="core", subcore_axis_name="subcore")

    @functools.partial(
        pl.kernel,
        out_type=jax.ShapeDtypeStruct((n, w), table.dtype),
        mesh=mesh,
    )
    def gather_kernel(table_hbm, idx_hbm, out_hbm):
        def body(idx_vmem, out_vmem):
            pltpu.sync_copy(table_hbm.at[idx_vmem.at[0]], out_vmem)

        pltpu.emit_pipeline(
            body,
            grid=(n // GW,),
            in_specs=[pl.BlockSpec((1, GW), lambda i: (0, i))],
            out_specs=[pl.BlockSpec((GW, w), lambda i: (i, 0))],
            core_axis_name=("core", "subcore"),
            dimension_semantics=(pltpu.PARALLEL,),
        )(idx_hbm, out_hbm)

    return gather_kernel(table, flat_idx)


def _mlp(x, w1, b1, w2, b2):
    k = x.shape[1]

    def body(x_ref, w1_ref, b1_ref, w2_ref, b2_ref, o_ref):
        xb = x_ref[...].astype(jnp.bfloat16)
        h = jnp.dot(xb, w1_ref[...], preferred_element_type=jnp.float32)
        h = jnp.maximum(h + b1_ref[...], 0.0).astype(jnp.bfloat16)
        o = jnp.dot(h, w2_ref[...], preferred_element_type=jnp.float32)
        o_ref[...] = o + b2_ref[...]

    return pl.pallas_call(
        body,
        grid=(B // BLK,),
        in_specs=[
            pl.BlockSpec((BLK, k), lambda i: (i, 0)),
            pl.BlockSpec((k, H), lambda i: (0, 0)),
            pl.BlockSpec((1, H), lambda i: (0, 0)),
            pl.BlockSpec((H, D), lambda i: (0, 0)),
            pl.BlockSpec((1, D), lambda i: (0, 0)),
        ],
        out_specs=pl.BlockSpec((BLK, D), lambda i: (i, 0)),
        out_shape=jax.ShapeDtypeStruct((B, D), jnp.float32),
    )(x, w1, b1.reshape(1, H), w2, b2.reshape(1, D))


def kernel(idx, tables, W1, b1, W2, b2):
    idx = idx.astype(jnp.int32)
    flat_idx = (idx + (jnp.arange(F, dtype=jnp.int32) * V)[None, :]).reshape(1, B * F)
    table = jnp.pad(tables.reshape(F * V, D), ((0, 0), (0, DP - D)))
    emb = _gather_rows(table, flat_idx)
    x = emb.reshape(B, F * DP)
    w1 = jnp.pad(
        W1.astype(jnp.bfloat16).reshape(F, D, H), ((0, 0), (0, DP - D), (0, 0))
    ).reshape(F * DP, H)
    return _mlp(x, w1, b1, W2.astype(jnp.bfloat16), b2)


# speedup vs baseline: 24.3672x; 1.6772x over previous
import functools

import jax
import jax.numpy as jnp
from jax.experimental import pallas as pl
from jax.experimental.pallas import tpu as pltpu
from jax.experimental.pallas import tpu_sc as plsc

B = 16384
F = 26
V = 1000
D = 64
H = 128
DP = 128

GW = 128
BLK = 256


def _gather_rows(table, flat_idx):
    n = flat_idx.shape[1]
    w = table.shape[1]
    mesh = plsc.VectorSubcoreMesh(core_axis_name="core", subcore_axis_name="subcore")

    @functools.partial(
        pl.kernel,
        out_type=jax.ShapeDtypeStruct((n, w), table.dtype),
        mesh=mesh,
    )
    def gather_kernel(table_hbm, idx_hbm, out_hbm):
        def body(idx_vmem, out_vmem):
            pltpu.sync_copy(table_hbm.at[idx_vmem.at[0]], out_vmem)

        pltpu.emit_pipeline(
            body,
            grid=(n // GW,),
            in_specs=[pl.BlockSpec((1, GW), lambda i: (0, i))],
            out_specs=[pl.BlockSpec((GW, w), lambda i: (i, 0))],
            core_axis_name=("core", "subcore"),
            dimension_semantics=(pltpu.PARALLEL,),
        )(idx_hbm, out_hbm)

    return gather_kernel(table, flat_idx)


def _mlp(x, w1, b1, w2, b2):
    def body(x_ref, w1_ref, b1_ref, w2_ref, b2_ref, o_ref):
        h = jnp.zeros((BLK, H), dtype=jnp.float32)
        for f in range(F):
            xb = x_ref[f].astype(jnp.bfloat16)
            h = h + jnp.dot(xb, w1_ref[f], preferred_element_type=jnp.float32)
        h = jnp.maximum(h + b1_ref[...], 0.0).astype(jnp.bfloat16)
        o = jnp.dot(h, w2_ref[...], preferred_element_type=jnp.float32)
        o_ref[...] = o + b2_ref[...]

    return pl.pallas_call(
        body,
        grid=(B // BLK,),
        in_specs=[
            pl.BlockSpec((F, BLK, DP), lambda i: (0, i, 0)),
            pl.BlockSpec((F, DP, H), lambda i: (0, 0, 0)),
            pl.BlockSpec((1, H), lambda i: (0, 0)),
            pl.BlockSpec((H, D), lambda i: (0, 0)),
            pl.BlockSpec((1, D), lambda i: (0, 0)),
        ],
        out_specs=pl.BlockSpec((BLK, D), lambda i: (i, 0)),
        out_shape=jax.ShapeDtypeStruct((B, D), jnp.float32),
    )(x, w1, b1.reshape(1, H), w2, b2.reshape(1, D))


def kernel(idx, tables, W1, b1, W2, b2):
    idx = idx.astype(jnp.int32)
    flat_idx = (idx.T + (jnp.arange(F, dtype=jnp.int32) * V)[:, None]).reshape(1, F * B)
    table = jnp.pad(tables.reshape(F * V, D), ((0, 0), (0, DP - D)))
    emb = _gather_rows(table, flat_idx)
    x = emb.reshape(F, B, DP)
    w1 = jnp.pad(
        W1.astype(jnp.bfloat16).reshape(F, D, H), ((0, 0), (0, DP - D), (0, 0))
    )
    return _mlp(x, w1, b1, W2.astype(jnp.bfloat16), b2)


# speedup vs baseline: 27.7656x; 1.1395x over previous
import functools

import jax
import jax.numpy as jnp
from jax.experimental import pallas as pl
from jax.experimental.pallas import tpu as pltpu
from jax.experimental.pallas import tpu_sc as plsc

B = 16384
F = 26
V = 1000
D = 64
H = 128
DP = 128

GW = 128
BLK = 256


NC = 2
NS = 16
NW = NC * NS


def _gather_rows(table, flat_idx):
    n = flat_idx.shape[0]
    w = table.shape[1]
    per_w = n // NW
    nwin = per_w // GW
    mesh = plsc.VectorSubcoreMesh(core_axis_name="core", subcore_axis_name="subcore")

    @functools.partial(
        pl.kernel,
        out_type=jax.ShapeDtypeStruct((n, w), jnp.float32),
        mesh=mesh,
        scratch_types=[
            pltpu.VMEM((per_w,), jnp.int32),
            pltpu.VMEM((GW, w), jnp.float32),
            pltpu.VMEM((GW, w), jnp.float32),
            pltpu.SemaphoreType.DMA,
            pltpu.SemaphoreType.DMA,
        ],
    )
    def gather_kernel(table_hbm, idx_hbm, out_hbm, idx_v, buf0, buf1, sem0, sem1):
        wid = jax.lax.axis_index("subcore") * NC + jax.lax.axis_index("core")
        base = wid * per_w
        pltpu.sync_copy(idx_hbm.at[pl.ds(base, per_w)], idx_v)

        def gather_start(win, buf, sem):
            pltpu.async_copy(table_hbm.at[idx_v.at[pl.ds(win * GW, GW)]], buf, sem)

        def drain_store(win, buf, sem):
            pltpu.make_async_copy(
                table_hbm.at[idx_v.at[pl.ds(win * GW, GW)]], buf, sem
            ).wait()
            pltpu.sync_copy(buf, out_hbm.at[pl.ds(base + win * GW, GW)])

        gather_start(0, buf0, sem0)

        @pl.loop(0, nwin, step=2)
        def _(wn):
            @pl.when(wn + 1 < nwin)
            def _():
                gather_start(wn + 1, buf1, sem1)

            drain_store(wn, buf0, sem0)

            @pl.when(wn + 2 < nwin)
            def _():
                gather_start(wn + 2, buf0, sem0)

            @pl.when(wn + 1 < nwin)
            def _():
                drain_store(wn + 1, buf1, sem1)

    return gather_kernel(table, flat_idx)


def _mlp(x, w1, b1, w2, b2):
    def body(x_ref, w1_ref, b1_ref, w2_ref, b2_ref, o_ref):
        h = jnp.zeros((BLK, H), dtype=jnp.float32)
        for f in range(F):
            xb = x_ref[f].astype(jnp.bfloat16)
            h = h + jnp.dot(xb, w1_ref[f], preferred_element_type=jnp.float32)
        h = jnp.maximum(h + b1_ref[...], 0.0).astype(jnp.bfloat16)
        o = jnp.dot(h, w2_ref[...], preferred_element_type=jnp.float32)
        o_ref[...] = o + b2_ref[...]

    return pl.pallas_call(
        body,
        grid=(B // BLK,),
        in_specs=[
            pl.BlockSpec((F, BLK, DP), lambda i: (0, i, 0)),
            pl.BlockSpec((F, DP, H), lambda i: (0, 0, 0)),
            pl.BlockSpec((1, H), lambda i: (0, 0)),
            pl.BlockSpec((H, D), lambda i: (0, 0)),
            pl.BlockSpec((1, D), lambda i: (0, 0)),
        ],
        out_specs=pl.BlockSpec((BLK, D), lambda i: (i, 0)),
        out_shape=jax.ShapeDtypeStruct((B, D), jnp.float32),
    )(x, w1, b1.reshape(1, H), w2, b2.reshape(1, D))


def kernel(idx, tables, W1, b1, W2, b2):
    idx = idx.astype(jnp.int32)
    flat_idx = (idx.T + (jnp.arange(F, dtype=jnp.int32) * V)[:, None]).reshape(F * B)
    table = jnp.pad(tables.reshape(F * V, D), ((0, 0), (0, DP - D)))
    emb = _gather_rows(table, flat_idx)
    x = emb.reshape(F, B, DP)
    w1 = jnp.pad(
        W1.astype(jnp.bfloat16).reshape(F, D, H), ((0, 0), (0, DP - D), (0, 0))
    )
    return _mlp(x, w1, b1, W2.astype(jnp.bfloat16), b2)


# speedup vs baseline: 28.0909x; 1.0117x over previous
import functools

import jax
import jax.numpy as jnp
from jax.experimental import pallas as pl
from jax.experimental.pallas import tpu as pltpu
from jax.experimental.pallas import tpu_sc as plsc

B = 16384
F = 26
V = 1000
D = 64
H = 128
DP = 128

GW = 128
BLK = 256


NC = 2
NS = 16
NW = NC * NS


def _gather_rows(table, flat_idx):
    n = flat_idx.shape[0]
    w = table.shape[1]
    per_w = n // NW
    nwin = per_w // GW
    mesh = plsc.VectorSubcoreMesh(core_axis_name="core", subcore_axis_name="subcore")

    nbuf = 4

    @functools.partial(
        pl.kernel,
        out_type=jax.ShapeDtypeStruct((n, w), jnp.float32),
        mesh=mesh,
        scratch_types=[pltpu.VMEM((per_w,), jnp.int32)]
        + [pltpu.VMEM((GW, w), jnp.float32)] * nbuf
        + [pltpu.SemaphoreType.DMA] * nbuf,
    )
    def gather_kernel(table_hbm, idx_hbm, out_hbm, idx_v, *rest):
        bufs, sems = rest[:nbuf], rest[nbuf:]
        wid = jax.lax.axis_index("subcore") * NC + jax.lax.axis_index("core")
        base = wid * per_w
        pltpu.sync_copy(idx_hbm.at[pl.ds(base, per_w)], idx_v)

        def gather_start(win, buf, sem):
            pltpu.async_copy(table_hbm.at[idx_v.at[pl.ds(win * GW, GW)]], buf, sem)

        def drain_store(win, buf, sem):
            pltpu.make_async_copy(
                table_hbm.at[idx_v.at[pl.ds(win * GW, GW)]], buf, sem
            ).wait()
            pltpu.sync_copy(buf, out_hbm.at[pl.ds(base + win * GW, GW)])

        for p in range(nbuf - 1):
            gather_start(p, bufs[p], sems[p])

        @pl.loop(0, nwin, step=nbuf)
        def _(wn):
            for j in range(nbuf):
                ahead = (j + nbuf - 1) % nbuf

                @pl.when(wn + j + nbuf - 1 < nwin)
                def _():
                    gather_start(wn + j + nbuf - 1, bufs[ahead], sems[ahead])

                drain_store(wn + j, bufs[j], sems[j])

    return gather_kernel(table, flat_idx)


def _mlp(x, w1, b1, w2, b2):
    def body(x_ref, w1_ref, b1_ref, w2_ref, b2_ref, o_ref):
        h = jnp.zeros((BLK, H), dtype=jnp.float32)
        for f in range(F):
            xb = x_ref[f].astype(jnp.bfloat16)
            h = h + jnp.dot(xb, w1_ref[f], preferred_element_type=jnp.float32)
        h = jnp.maximum(h + b1_ref[...], 0.0).astype(jnp.bfloat16)
        o = jnp.dot(h, w2_ref[...], preferred_element_type=jnp.float32)
        o_ref[...] = o + b2_ref[...]

    return pl.pallas_call(
        body,
        grid=(B // BLK,),
        in_specs=[
            pl.BlockSpec((F, BLK, DP), lambda i: (0, i, 0)),
            pl.BlockSpec((F, DP, H), lambda i: (0, 0, 0)),
            pl.BlockSpec((1, H), lambda i: (0, 0)),
            pl.BlockSpec((H, D), lambda i: (0, 0)),
            pl.BlockSpec((1, D), lambda i: (0, 0)),
        ],
        out_specs=pl.BlockSpec((BLK, D), lambda i: (i, 0)),
        out_shape=jax.ShapeDtypeStruct((B, D), jnp.float32),
    )(x, w1, b1.reshape(1, H), w2, b2.reshape(1, D))


def kernel(idx, tables, W1, b1, W2, b2):
    idx = idx.astype(jnp.int32)
    flat_idx = (idx.T + (jnp.arange(F, dtype=jnp.int32) * V)[:, None]).reshape(F * B)
    table = jnp.pad(tables.reshape(F * V, D), ((0, 0), (0, DP - D)))
    emb = _gather_rows(table, flat_idx)
    x = emb.reshape(F, B, DP)
    w1 = jnp.pad(
        W1.astype(jnp.bfloat16).reshape(F, D, H), ((0, 0), (0, DP - D), (0, 0))
    )
    return _mlp(x, w1, b1, W2.astype(jnp.bfloat16), b2)
